# baseline (speedup 1.0000x reference)
import functools

import jax
import jax.numpy as jnp
from jax import lax
from jax.experimental import pallas as pl
from jax.experimental.pallas import tpu as pltpu
from jax.experimental.pallas import tpu_sc as plsc

N = 10000
K = 32
DM = 128
DE = 32
NG = 32
TBL = 128
TCO = 128
NK = N * K

SC_NC = 2
SC_NS = 16
SC_NW = SC_NC * SC_NS
EPW = NK // SC_NW
CH = 80
NCHUNK = EPW // CH


def _ln(x, g, b, eps=1e-5):
    m = jnp.mean(x, axis=-1, keepdims=True)
    v = jnp.mean((x - m) ** 2, axis=-1, keepdims=True)
    return (x - m) * lax.rsqrt(v + eps) * g + b


BA = 1000


def _pre_body(te, ge, lng, lnb, qkvW, qkvb, W1q, W1k, W1g, mab1,
              dstA_ref, table_ref):
    x = te[:]
    xn = _ln(x, lng[:], lnb[:])
    qkv = jnp.dot(xn, qkvW[:], preferred_element_type=jnp.float32) + qkvb[:]
    q = qkv[:, :DM]
    kk = qkv[:, DM:2 * DM]
    vv = qkv[:, 2 * DM:]
    gW = jnp.dot(ge[:], W1g[:], preferred_element_type=jnp.float32)
    dstA_ref[:] = jnp.dot(q, W1q[:], preferred_element_type=jnp.float32) - gW + mab1[:]
    srcB = jnp.dot(kk, W1k[:], preferred_element_type=jnp.float32) + gW
    def rne_hi16(x):
        u = lax.bitcast_convert_type(x, jnp.int32)
        return lax.shift_right_logical(
            u + 0x7FFF + (lax.shift_right_logical(u, 16) & 1), 16)
    packed = lax.shift_left(rne_hi16(srcB), 16) | rne_hi16(vv)
    table_ref[:] = lax.bitcast_convert_type(packed, jnp.float32)


def _precompute(te, ge, lng, lnb, qkvW, qkvb, W1q, W1k, W1g, mab1):
    full = lambda shape: pl.BlockSpec(shape, lambda i: (0, 0))
    row = lambda shape: pl.BlockSpec(shape, lambda i: (i, 0))
    return pl.pallas_call(
        _pre_body,
        grid=(N // BA,),
        in_specs=[
            row((BA, DM)), row((BA, NG)),
            full((1, DM)), full((1, DM)), full((DM, 3 * DM)), full((1, 3 * DM)),
            full((DM, DM)), full((DM, DM)), full((NG, DM)), full((1, DM)),
        ],
        out_specs=[row((BA, DM)), row((BA, TBL))],
        out_shape=[
            jax.ShapeDtypeStruct((N, DM), jnp.float32),
            jax.ShapeDtypeStruct((N, TBL), jnp.float32),
        ],
    )(te, ge, lng, lnb, qkvW, qkvb, W1q, W1k, W1g, mab1)


def _sc_gather(idx, table, cot, nk, ch):
    epw = nk // SC_NW
    nchunk = epw // ch
    mesh = plsc.VectorSubcoreMesh(core_axis_name="c", subcore_axis_name="s")

    @functools.partial(
        pl.kernel, mesh=mesh,
        out_type=(
            jax.ShapeDtypeStruct((nk, TBL), jnp.float32),
            jax.ShapeDtypeStruct((nk, TCO), jnp.float32),
        ),
        scratch_types=[
            pltpu.VMEM((2, ch), jnp.int32),
            pltpu.VMEM((2, ch, TBL), jnp.float32),
            pltpu.VMEM((2, ch, TCO), jnp.float32),
            pltpu.SemaphoreType.DMA,
            pltpu.SemaphoreType.DMA,
            pltpu.SemaphoreType.DMA,
        ],
    )
    def gk(idx_hbm, table_hbm, cot_hbm, out_hbm, outco_hbm,
           idx_v, rows_v, rco_v, semg, semw0, semw1):
        wid = lax.axis_index("s") * SC_NC + lax.axis_index("c")
        w0 = wid * epw
        semw = (semw0, semw1)

        def drain_write(b):
            pltpu.make_async_copy(
                out_hbm.at[pl.ds(0, ch)], rows_v.at[b], semw[b]).wait()
            pltpu.make_async_copy(
                outco_hbm.at[pl.ds(0, ch)], rco_v.at[b], semw[b]).wait()

        def chunk(i, b, wait_prev):
            base = w0 + i * ch
            pltpu.sync_copy(idx_hbm.at[pl.ds(base, ch)], idx_v.at[b])
            if wait_prev:
                drain_write(b)
            cp1 = pltpu.async_copy(table_hbm.at[idx_v.at[b]], rows_v.at[b], semg)
            cp2 = pltpu.async_copy(cot_hbm.at[idx_v.at[b]], rco_v.at[b], semg)
            cp1.wait()
            cp2.wait()
            pltpu.async_copy(rows_v.at[b], out_hbm.at[pl.ds(base, ch)], semw[b])
            pltpu.async_copy(rco_v.at[b], outco_hbm.at[pl.ds(base, ch)], semw[b])

        chunk(0, 0, False)
        chunk(1, 1, False)

        def pair(p, carry):
            chunk(2 * p, 0, True)
            chunk(2 * p + 1, 1, True)
            return carry

        lax.fori_loop(1, nchunk // 2, pair, 0)
        if nchunk % 2:
            chunk(nchunk - 1, 0, True)
        drain_write(0)
        drain_write(1)

    return gk(idx, table, cot)


BN = 200


def _edge_body(gath, gco, dstA, co16,
               etw10_4, etw11_4, etw12_4, etb1_4, etg_4, etbn_4,
               mavg, w2s4, etb2,
               maW1f, gw2, sgw2,
               woW1, wob1, wog, wobn, woW2, wob2,
               out_ref):
    gi = lax.bitcast_convert_type(gath[:], jnp.int32).reshape(BN, K, TBL)
    srcB = lax.bitcast_convert_type(gi & jnp.int32(-65536), jnp.float32)
    vv = lax.bitcast_convert_type(lax.shift_left(gi, 16), jnp.float32)
    c3 = gco[:].reshape(BN, K, TCO)
    nx = c3[:, :, 0:1]
    ny = c3[:, :, 1:2]
    dx = co16[:, 0:1].reshape(BN, 1, 1)
    dy = co16[:, 1:2].reshape(BN, 1, 1)
    rx = nx - dx
    ry = ny - dy
    rnorm = jnp.sqrt(rx * rx + ry * ry)
    cx = jnp.mean(rx, axis=1, keepdims=True)
    cy = jnp.mean(ry, axis=1, keepdims=True)
    xc = rx - cx
    yc = ry - cy
    cxx = jnp.sum(xc * xc, axis=1, keepdims=True)
    cxy = jnp.sum(xc * yc, axis=1, keepdims=True)
    cyy = jnp.sum(yc * yc, axis=1, keepdims=True)
    half = 0.5 * (cxx - cyy)
    r = jnp.sqrt(half * half + cxy * cxy)
    ax = r - half
    ay = r + half
    use1 = jnp.abs(ax) > jnp.abs(ay)
    v1x = jnp.where(use1, cxy, ay)
    v1y = jnp.where(use1, ax, cxy)
    nrm = jnp.sqrt(v1x * v1x + v1y * v1y)
    ok = nrm > 1e-30
    inv = lax.rsqrt(jnp.where(ok, nrm * nrm, 1.0))
    v1x = jnp.where(ok, v1x * inv, 0.0)
    v1y = jnp.where(ok, v1y * inv, 1.0)
    p0 = -xc * v1y + yc * v1x
    p1 = xc * v1x + yc * v1y
    hin = (rnorm * etw12_4[:].reshape(1, 1, 4 * DE)
           + etb1_4[:].reshape(1, 1, 4 * DE)
           + p0 * etw10_4[:].reshape(1, 1, 4 * DE)
           + p1 * etw11_4[:].reshape(1, 1, 4 * DE))
    a = jax.nn.gelu(hin).reshape(BN * K, 4 * DE)
    mg = jnp.dot(a, mavg[:], preferred_element_type=jnp.float32)
    sg = jnp.dot(a * a, mavg[:], preferred_element_type=jnp.float32)
    var = jnp.maximum(sg - mg * mg, 0.0)
    aN = (a - mg) * lax.rsqrt(var + 1e-5) * etg_4[:] + etbn_4[:]
    ef = jnp.dot(aN, w2s4[:], preferred_element_type=jnp.float32) + etb2[:]
    fh = ef.reshape(BN, K, DE)
    h1 = jnp.dot(ef, maW1f[:], preferred_element_type=jnp.float32).reshape(BN, K, DM)
    h1 = h1 + dstA[:][:, None, :] + srcB
    a1 = jax.nn.gelu(h1)
    m1 = jnp.mean(a1, axis=-1, keepdims=True)
    q1 = jnp.mean(a1 * a1, axis=-1, keepdims=True)
    var1 = jnp.maximum(q1 - m1 * m1, 0.0)
    t1 = jnp.sum(a1 * gw2[:].reshape(1, 1, DM), axis=-1, keepdims=True)
    logits = (t1 - m1 * sgw2[:, 0:1].reshape(1, 1, 1)) * lax.rsqrt(var1 + 1e-5)
    mx = jnp.max(logits, axis=1, keepdims=True)
    e = jnp.exp(logits - mx)
    attn = e / jnp.sum(e, axis=1, keepdims=True)
    node = jnp.sum(attn * vv, axis=1)
    edge = jnp.sum(attn * fh, axis=1)
    cat = jnp.concatenate([node, edge], axis=-1)
    h = jnp.dot(cat, woW1[:], preferred_element_type=jnp.float32) + wob1[:]
    h = _ln(jax.nn.gelu(h), wog[:], wobn[:])
    out_ref[:] = jnp.dot(h, woW2[:], preferred_element_type=jnp.float32) + wob2[:]


def _edge_compute(gath, gco, dstA, co16, weights):
    nn = gath.shape[0] // K
    full = lambda shape: pl.BlockSpec(shape, lambda i: (0, 0))
    row = lambda shape: pl.BlockSpec(shape, lambda i: (i, 0))
    w_specs = [full(w.shape) for w in weights]
    return pl.pallas_call(
        _edge_body,
        grid=(nn // BN,),
        in_specs=[row((BN * K, TBL)), row((BN * K, TCO)), row((BN, DM)),
                  row((BN, 16))] + w_specs,
        out_specs=row((BN, DM)),
        out_shape=jax.ShapeDtypeStruct((nn, DM), jnp.float32),
    )(gath, gco, dstA, co16, *weights)


def kernel(gene_exp, token_embs, coords, params, neighbor_indices):
    p = params
    co16 = jnp.pad(coords, ((0, 0), (0, 14)))
    cot = jnp.pad(coords, ((0, 0), (0, TCO - 2)))
    maW1 = p['ma_W1']
    W1q = maW1[:DM]
    W1k = maW1[DM:2 * DM]
    W1f = maW1[2 * DM:2 * DM + DE]
    W1g = maW1[2 * DM + DE:]
    r2 = lambda a: a.reshape(1, -1)
    dstA, table = _precompute(
        token_embs, gene_exp,
        r2(p['ln_qkv_g']), r2(p['ln_qkv_b']), p['qkv_W'], r2(p['qkv_b']),
        W1q, W1k, W1g, r2(p['ma_b1']))
    idx = neighbor_indices.reshape(-1).astype(jnp.int32)
    S = 10
    ek = NK // S
    parts = [_sc_gather(idx[s * ek:(s + 1) * ek], table, cot, ek, 40)
             for s in range(S)]
    etW1 = p['et_W1']
    cat4 = lambda s: jnp.concatenate(s, axis=0).reshape(1, -1)
    w10, w11, w12 = etW1[0], etW1[1], etW1[2]
    etw10_4 = cat4([w10, w10, -w10, -w10])
    etw11_4 = cat4([w11, -w11, w11, -w11])
    etw12_4 = cat4([w12] * 4)
    etb1_4 = cat4([p['et_b1']] * 4)
    etg_4 = cat4([p['et_g']] * 4)
    etbn_4 = cat4([p['et_bn']] * 4)
    eye4 = jnp.eye(4, dtype=jnp.float32)
    mavg = jnp.kron(eye4, jnp.full((DE, DE), 1.0 / DE, jnp.float32))
    w2s4 = jnp.concatenate([p['et_W2']] * 4, axis=0) * 0.25
    gw2 = (p['ma_g'] * p['ma_W2'][:, 0]).reshape(1, -1)
    sgw2 = jnp.broadcast_to(jnp.sum(gw2), (1, 128))
    weights = [
        etw10_4, etw11_4, etw12_4, etb1_4, etg_4, etbn_4,
        mavg, w2s4, r2(p['et_b2']),
        W1f, gw2, sgw2,
        p['wo_W1'], r2(p['wo_b1']), r2(p['wo_g']), r2(p['wo_bn']),
        p['wo_W2'], r2(p['wo_b2']),
    ]
    en = N // S
    outs = [_edge_compute(g, c, dstA[s * en:(s + 1) * en],
                          co16[s * en:(s + 1) * en], weights)
            for s, (g, c) in enumerate(parts)]
    return jnp.concatenate(outs, axis=0)


# speedup vs baseline: 1.0094x; 1.0094x over previous
import functools

import jax
import jax.numpy as jnp
from jax import lax
from jax.experimental import pallas as pl
from jax.experimental.pallas import tpu as pltpu
from jax.experimental.pallas import tpu_sc as plsc

N = 10000
K = 32
DM = 128
DE = 32
NG = 32
TBL = 128
TCO = 128
NK = N * K

SC_NC = 2
SC_NS = 16
SC_NW = SC_NC * SC_NS
EPW = NK // SC_NW
CH = 80
NCHUNK = EPW // CH


def _ln(x, g, b, eps=1e-5):
    m = jnp.mean(x, axis=-1, keepdims=True)
    v = jnp.mean((x - m) ** 2, axis=-1, keepdims=True)
    return (x - m) * lax.rsqrt(v + eps) * g + b


BA = 1000


def _pre_body(te, ge, lng, lnb, qkvW, qkvb, W1q, W1k, W1g, mab1,
              dstA_ref, table_ref):
    x = te[:]
    xn = _ln(x, lng[:], lnb[:])
    qkv = jnp.dot(xn, qkvW[:], preferred_element_type=jnp.float32) + qkvb[:]
    q = qkv[:, :DM]
    kk = qkv[:, DM:2 * DM]
    vv = qkv[:, 2 * DM:]
    gW = jnp.dot(ge[:], W1g[:], preferred_element_type=jnp.float32)
    dstA_ref[:] = jnp.dot(q, W1q[:], preferred_element_type=jnp.float32) - gW + mab1[:]
    srcB = jnp.dot(kk, W1k[:], preferred_element_type=jnp.float32) + gW
    def rne_hi16(x):
        u = lax.bitcast_convert_type(x, jnp.int32)
        return lax.shift_right_logical(
            u + 0x7FFF + (lax.shift_right_logical(u, 16) & 1), 16)
    packed = lax.shift_left(rne_hi16(srcB), 16) | rne_hi16(vv)
    table_ref[:] = lax.bitcast_convert_type(packed, jnp.float32)


def _precompute(te, ge, lng, lnb, qkvW, qkvb, W1q, W1k, W1g, mab1):
    full = lambda shape: pl.BlockSpec(shape, lambda i: (0, 0))
    row = lambda shape: pl.BlockSpec(shape, lambda i: (i, 0))
    return pl.pallas_call(
        _pre_body,
        grid=(N // BA,),
        in_specs=[
            row((BA, DM)), row((BA, NG)),
            full((1, DM)), full((1, DM)), full((DM, 3 * DM)), full((1, 3 * DM)),
            full((DM, DM)), full((DM, DM)), full((NG, DM)), full((1, DM)),
        ],
        out_specs=[row((BA, DM)), row((BA, TBL))],
        out_shape=[
            jax.ShapeDtypeStruct((N, DM), jnp.float32),
            jax.ShapeDtypeStruct((N, TBL), jnp.float32),
        ],
    )(te, ge, lng, lnb, qkvW, qkvb, W1q, W1k, W1g, mab1)


def _sc_gather(idx, table, cot, nk, ch):
    epw = nk // SC_NW
    nchunk = epw // ch
    mesh = plsc.VectorSubcoreMesh(core_axis_name="c", subcore_axis_name="s")

    @functools.partial(
        pl.kernel, mesh=mesh,
        out_type=(
            jax.ShapeDtypeStruct((nk, TBL), jnp.float32),
            jax.ShapeDtypeStruct((nk, TCO), jnp.float32),
        ),
        scratch_types=[
            pltpu.VMEM((2, ch), jnp.int32),
            pltpu.VMEM((2, ch, TBL), jnp.float32),
            pltpu.VMEM((2, ch, TCO), jnp.float32),
            pltpu.SemaphoreType.DMA,
            pltpu.SemaphoreType.DMA,
            pltpu.SemaphoreType.DMA,
        ],
    )
    def gk(idx_hbm, table_hbm, cot_hbm, out_hbm, outco_hbm,
           idx_v, rows_v, rco_v, semg, semw0, semw1):
        wid = lax.axis_index("s") * SC_NC + lax.axis_index("c")
        w0 = wid * epw
        semw = (semw0, semw1)

        def drain_write(b):
            pltpu.make_async_copy(
                out_hbm.at[pl.ds(0, ch)], rows_v.at[b], semw[b]).wait()
            pltpu.make_async_copy(
                outco_hbm.at[pl.ds(0, ch)], rco_v.at[b], semw[b]).wait()

        def chunk(i, b, wait_prev):
            base = w0 + i * ch
            pltpu.sync_copy(idx_hbm.at[pl.ds(base, ch)], idx_v.at[b])
            if wait_prev:
                drain_write(b)
            cp1 = pltpu.async_copy(table_hbm.at[idx_v.at[b]], rows_v.at[b], semg)
            cp2 = pltpu.async_copy(cot_hbm.at[idx_v.at[b]], rco_v.at[b], semg)
            cp1.wait()
            cp2.wait()
            pltpu.async_copy(rows_v.at[b], out_hbm.at[pl.ds(base, ch)], semw[b])
            pltpu.async_copy(rco_v.at[b], outco_hbm.at[pl.ds(base, ch)], semw[b])

        chunk(0, 0, False)
        chunk(1, 1, False)

        def pair(p, carry):
            chunk(2 * p, 0, True)
            chunk(2 * p + 1, 1, True)
            return carry

        lax.fori_loop(1, nchunk // 2, pair, 0)
        if nchunk % 2:
            chunk(nchunk - 1, 0, True)
        drain_write(0)
        drain_write(1)

    return gk(idx, table, cot)


BN = 200


def _edge_body(gath, gco, dstA, co16,
               etw10_4, etw11_4, etw12_4, etb1_4, etg_4, etbn_4,
               mavg, w2s4, etb2,
               maW1f, gw2, sgw2,
               woW1, wob1, wog, wobn, woW2, wob2,
               out_ref):
    gi = lax.bitcast_convert_type(gath[:], jnp.int32).reshape(BN, K, TBL)
    srcB = lax.bitcast_convert_type(gi & jnp.int32(-65536), jnp.float32)
    vv = lax.bitcast_convert_type(lax.shift_left(gi, 16), jnp.float32)
    c3 = gco[:].reshape(BN, K, TCO)
    nx = c3[:, :, 0:1]
    ny = c3[:, :, 1:2]
    dx = co16[:, 0:1].reshape(BN, 1, 1)
    dy = co16[:, 1:2].reshape(BN, 1, 1)
    rx = nx - dx
    ry = ny - dy
    rnorm = jnp.sqrt(rx * rx + ry * ry)
    cx = jnp.mean(rx, axis=1, keepdims=True)
    cy = jnp.mean(ry, axis=1, keepdims=True)
    xc = rx - cx
    yc = ry - cy
    cxx = jnp.sum(xc * xc, axis=1, keepdims=True)
    cxy = jnp.sum(xc * yc, axis=1, keepdims=True)
    cyy = jnp.sum(yc * yc, axis=1, keepdims=True)
    half = 0.5 * (cxx - cyy)
    r = jnp.sqrt(half * half + cxy * cxy)
    ax = r - half
    ay = r + half
    use1 = jnp.abs(ax) > jnp.abs(ay)
    v1x = jnp.where(use1, cxy, ay)
    v1y = jnp.where(use1, ax, cxy)
    nrm = jnp.sqrt(v1x * v1x + v1y * v1y)
    ok = nrm > 1e-30
    inv = lax.rsqrt(jnp.where(ok, nrm * nrm, 1.0))
    v1x = jnp.where(ok, v1x * inv, 0.0)
    v1y = jnp.where(ok, v1y * inv, 1.0)
    p0 = -xc * v1y + yc * v1x
    p1 = xc * v1x + yc * v1y
    hin = (rnorm * etw12_4[:].reshape(1, 1, 4 * DE)
           + etb1_4[:].reshape(1, 1, 4 * DE)
           + p0 * etw10_4[:].reshape(1, 1, 4 * DE)
           + p1 * etw11_4[:].reshape(1, 1, 4 * DE))
    a = jax.nn.gelu(hin).reshape(BN * K, 4 * DE)
    mg = jnp.dot(a, mavg[:], preferred_element_type=jnp.float32)
    sg = jnp.dot(a * a, mavg[:], preferred_element_type=jnp.float32)
    var = jnp.maximum(sg - mg * mg, 0.0)
    aN = (a - mg) * lax.rsqrt(var + 1e-5) * etg_4[:] + etbn_4[:]
    ef = jnp.dot(aN, w2s4[:], preferred_element_type=jnp.float32) + etb2[:]
    fh = ef.reshape(BN, K, DE)
    h1 = jnp.dot(ef, maW1f[:], preferred_element_type=jnp.float32).reshape(BN, K, DM)
    h1 = h1 + dstA[:][:, None, :] + srcB
    a1 = jax.nn.gelu(h1)
    m1 = jnp.mean(a1, axis=-1, keepdims=True)
    q1 = jnp.mean(a1 * a1, axis=-1, keepdims=True)
    var1 = jnp.maximum(q1 - m1 * m1, 0.0)
    t1 = jnp.sum(a1 * gw2[:].reshape(1, 1, DM), axis=-1, keepdims=True)
    logits = (t1 - m1 * sgw2[:, 0:1].reshape(1, 1, 1)) * lax.rsqrt(var1 + 1e-5)
    mx = jnp.max(logits, axis=1, keepdims=True)
    e = jnp.exp(logits - mx)
    attn = e / jnp.sum(e, axis=1, keepdims=True)
    node = jnp.sum(attn * vv, axis=1)
    edge = jnp.sum(attn * fh, axis=1)
    cat = jnp.concatenate([node, edge], axis=-1)
    h = jnp.dot(cat, woW1[:], preferred_element_type=jnp.float32) + wob1[:]
    h = _ln(jax.nn.gelu(h), wog[:], wobn[:])
    out_ref[:] = jnp.dot(h, woW2[:], preferred_element_type=jnp.float32) + wob2[:]


def _edge_compute(gath, gco, dstA, co16, weights):
    nn = gath.shape[0] // K
    full = lambda shape: pl.BlockSpec(shape, lambda i: (0, 0))
    row = lambda shape: pl.BlockSpec(shape, lambda i: (i, 0))
    w_specs = [full(w.shape) for w in weights]
    return pl.pallas_call(
        _edge_body,
        grid=(nn // BN,),
        in_specs=[row((BN * K, TBL)), row((BN * K, TCO)), row((BN, DM)),
                  row((BN, 16))] + w_specs,
        out_specs=row((BN, DM)),
        out_shape=jax.ShapeDtypeStruct((nn, DM), jnp.float32),
    )(gath, gco, dstA, co16, *weights)


def kernel(gene_exp, token_embs, coords, params, neighbor_indices):
    p = params
    co16 = jnp.pad(coords, ((0, 0), (0, 14)))
    cot = jnp.pad(coords, ((0, 0), (0, TCO - 2)))
    maW1 = p['ma_W1']
    W1q = maW1[:DM]
    W1k = maW1[DM:2 * DM]
    W1f = maW1[2 * DM:2 * DM + DE]
    W1g = maW1[2 * DM + DE:]
    r2 = lambda a: a.reshape(1, -1)
    dstA, table = _precompute(
        token_embs, gene_exp,
        r2(p['ln_qkv_g']), r2(p['ln_qkv_b']), p['qkv_W'], r2(p['qkv_b']),
        W1q, W1k, W1g, r2(p['ma_b1']))
    idx = neighbor_indices.reshape(-1).astype(jnp.int32)
    S = 5
    ek = NK // S
    parts = [_sc_gather(idx[s * ek:(s + 1) * ek], table, cot, ek, 80)
             for s in range(S)]
    etW1 = p['et_W1']
    cat4 = lambda s: jnp.concatenate(s, axis=0).reshape(1, -1)
    w10, w11, w12 = etW1[0], etW1[1], etW1[2]
    etw10_4 = cat4([w10, w10, -w10, -w10])
    etw11_4 = cat4([w11, -w11, w11, -w11])
    etw12_4 = cat4([w12] * 4)
    etb1_4 = cat4([p['et_b1']] * 4)
    etg_4 = cat4([p['et_g']] * 4)
    etbn_4 = cat4([p['et_bn']] * 4)
    eye4 = jnp.eye(4, dtype=jnp.float32)
    mavg = jnp.kron(eye4, jnp.full((DE, DE), 1.0 / DE, jnp.float32))
    w2s4 = jnp.concatenate([p['et_W2']] * 4, axis=0) * 0.25
    gw2 = (p['ma_g'] * p['ma_W2'][:, 0]).reshape(1, -1)
    sgw2 = jnp.broadcast_to(jnp.sum(gw2), (1, 128))
    weights = [
        etw10_4, etw11_4, etw12_4, etb1_4, etg_4, etbn_4,
        mavg, w2s4, r2(p['et_b2']),
        W1f, gw2, sgw2,
        p['wo_W1'], r2(p['wo_b1']), r2(p['wo_g']), r2(p['wo_bn']),
        p['wo_W2'], r2(p['wo_b2']),
    ]
    en = N // S
    outs = [_edge_compute(g, c, dstA[s * en:(s + 1) * en],
                          co16[s * en:(s + 1) * en], weights)
            for s, (g, c) in enumerate(parts)]
    return jnp.concatenate(outs, axis=0)
